# baseline (speedup 1.0000x reference)
import functools

import jax
import jax.numpy as jnp
from jax import lax
from jax.experimental import pallas as pl
from jax.experimental.pallas import tpu as pltpu
from jax.experimental.pallas import tpu_sc as plsc

N_NODES = 10000
N_EDGES = 320000
N_GRAPHS = 64
D_IN = 128
H = 200
HH = 104
HHV = H // 2
N_LAYERS = 5

_F32 = jnp.float32
_PREC = jax.lax.Precision.HIGHEST

_NS = 16
_EPT = N_EDGES // _NS
_K = 100
_NCH = _EPT // _K
_SCH = 20
_NSLAB = _NCH // _SCH
_NBUF = 5
_PAD_NODES = 10112
_RPT = _PAD_NODES // _NS


def _dot(a, b, precision=None):
    return lax.dot_general(a, b, (((1,), (0,)), ((), ())),
                           precision=precision, preferred_element_type=_F32)


def _sc_segsum_body(m0_hbm, m1_hbm, e_hbm, z_hbm, o0_hbm, o1_hbm,
                    islab0, islab1, rows0, rows1, rows2, rows3, rows4, acc,
                    isem0, isem1, gsem0, gsem1, gsem2, gsem3, gsem4,
                    ssem0, ssem1, ssem2, ssem3, ssem4):
    c = lax.axis_index("c")
    s = lax.axis_index("s")
    ib = (islab0, islab1)
    isem = (isem0, isem1)
    rows = (rows0, rows1, rows2, rows3, rows4)
    gs = (gsem0, gsem1, gsem2, gsem3, gsem4)
    ss = (ssem0, ssem1, ssem2, ssem3, ssem4)

    def fire_islab(u):
        pltpu.async_copy(e_hbm.at[s, u], ib[u % 2], isem[u % 2])

    def wait_islab(u):
        pltpu.make_async_copy(e_hbm.at[s, u], ib[u % 2], isem[u % 2]).wait()

    pltpu.sync_copy(z_hbm, acc.at[pl.ds(s * _RPT, _RPT)])
    fire_islab(0)
    fire_islab(1)
    plsc.subcore_barrier()

    def _half(m_hbm, o_hbm):
        def fire_gather(buf, kk, b):
            pltpu.async_copy(m_hbm.at[buf.at[kk, 0]], rows[b], gs[b])

        def wait_gather(buf, kk, b):
            pltpu.make_async_copy(m_hbm.at[buf.at[kk, 0]], rows[b],
                                  gs[b]).wait()

        def fire_scatter(buf, kk, b):
            pltpu.async_copy(rows[b], acc.at[buf.at[kk, 1]], ss[b], add=True)

        def wait_scatter(buf, kk, b):
            pltpu.make_async_copy(rows[b], acc.at[buf.at[kk, 1]],
                                  ss[b]).wait()

        wait_islab(0)
        fire_gather(ib[0], 0, 0)
        fire_gather(ib[0], 1, 1)

        for u in range(_NSLAB):
            bu = ib[u % 2]

            def body(k, carry, u=u, bu=bu):
                kk0 = _NBUF * k
                for b in range(_NBUF):
                    kk = kk0 + b
                    b2 = (b + 2) % _NBUF
                    wait_gather(bu, kk, b)
                    fire_scatter(bu, kk, b)
                    if u == 0 and b < 3:
                        @pl.when(k > 0)
                        def _():
                            wait_scatter(bu, kk, b2)
                    else:
                        wait_scatter(bu, kk, b2)
                    if b < _NBUF - 2:
                        fire_gather(bu, kk + 2, b2)
                    else:
                        @pl.when(k < _SCH // _NBUF - 1)
                        def _():
                            fire_gather(bu, kk + 2, b2)
                    if b == _NBUF - 1 and 1 <= u <= _NSLAB - 2:
                        @pl.when(k == 1)
                        def _():
                            fire_islab(u + 1)
                return carry

            lax.fori_loop(0, _SCH // _NBUF, body, 0)

            if u < _NSLAB - 1:
                nb = ib[(u + 1) % 2]
                wait_islab(u + 1)
                fire_gather(nb, 0, 0)
                fire_gather(nb, 1, 1)

        lastb = ib[(_NSLAB - 1) % 2]
        wait_scatter(lastb, _SCH - 3, (_SCH - 3) % _NBUF)
        wait_scatter(lastb, _SCH - 2, (_SCH - 2) % _NBUF)
        wait_scatter(lastb, _SCH - 1, (_SCH - 1) % _NBUF)
        plsc.subcore_barrier()
        pltpu.sync_copy(acc.at[pl.ds(s * _RPT, _RPT)],
                        o_hbm.at[pl.ds(s * _RPT, _RPT)])

    @pl.when(c == 0)
    def _():
        _half(m0_hbm, o0_hbm)

    @pl.when(c == 1)
    def _():
        _half(m1_hbm, o1_hbm)


_sc_segsum = functools.partial(
    pl.kernel,
    mesh=plsc.VectorSubcoreMesh(core_axis_name="c", subcore_axis_name="s"),
    compiler_params=pltpu.CompilerParams(use_tc_tiling_on_sc=False),
    out_type=[
        jax.ShapeDtypeStruct((_PAD_NODES, HH), _F32),
        jax.ShapeDtypeStruct((_PAD_NODES, HH), _F32),
    ],
    scratch_types=[
        pltpu.VMEM((_SCH, 2, _K), jnp.int32),
        pltpu.VMEM((_SCH, 2, _K), jnp.int32),
        pltpu.VMEM((_K, HH), _F32),
        pltpu.VMEM((_K, HH), _F32),
        pltpu.VMEM((_K, HH), _F32),
        pltpu.VMEM((_K, HH), _F32),
        pltpu.VMEM((_K, HH), _F32),
        pltpu.VMEM_SHARED((_PAD_NODES, HH), _F32),
    ] + [pltpu.SemaphoreType.DMA] * 12,
)(_sc_segsum_body)


_RB = 1000


def _embed_body(x_ref, w_ref, b_ref, o_ref):
    o_ref[...] = _dot(x_ref[...], w_ref[...]) + b_ref[...]


_embed = pl.pallas_call(
    _embed_body,
    grid=(N_NODES // _RB,),
    in_specs=[
        pl.BlockSpec((_RB, D_IN), lambda i: (i, 0)),
        pl.BlockSpec((D_IN, H), lambda i: (0, 0)),
        pl.BlockSpec((1, H), lambda i: (0, 0)),
    ],
    out_specs=pl.BlockSpec((_RB, H), lambda i: (i, 0)),
    out_shape=jax.ShapeDtypeStruct((N_NODES, H), _F32),
)


def _mm2_body(h_ref, wg0_ref, bg0_ref, wg1_ref, bg1_ref, wr_ref, br_ref,
              m0_ref, m1_ref, r_ref):
    h = h_ref[...]
    m0_ref[...] = _dot(h, wg0_ref[...]) + bg0_ref[...]
    m1_ref[...] = _dot(h, wg1_ref[...]) + bg1_ref[...]
    r_ref[...] = _dot(h, wr_ref[...]) + br_ref[...]


_mm2 = pl.pallas_call(
    _mm2_body,
    grid=(N_NODES // _RB,),
    in_specs=[
        pl.BlockSpec((_RB, H), lambda i: (i, 0)),
        pl.BlockSpec((H, HH), lambda i: (0, 0)),
        pl.BlockSpec((1, HH), lambda i: (0, 0)),
        pl.BlockSpec((H, HH), lambda i: (0, 0)),
        pl.BlockSpec((1, HH), lambda i: (0, 0)),
        pl.BlockSpec((H, H), lambda i: (0, 0)),
        pl.BlockSpec((1, H), lambda i: (0, 0)),
    ],
    out_specs=[
        pl.BlockSpec((_RB, HH), lambda i: (i, 0)),
        pl.BlockSpec((_RB, HH), lambda i: (i, 0)),
        pl.BlockSpec((_RB, H), lambda i: (i, 0)),
    ],
    out_shape=[
        jax.ShapeDtypeStruct((N_NODES, HH), _F32),
        jax.ShapeDtypeStruct((N_NODES, HH), _F32),
        jax.ShapeDtypeStruct((N_NODES, H), _F32),
    ],
)


def _post_body(p0_ref, p1_ref, r_ref, t_ref, stats_ref):
    j = pl.program_id(0)
    a = jnp.concatenate([p0_ref[:, :HHV], p1_ref[:, :HHV]], axis=1)
    t = jnp.maximum(a, 0.0) + jnp.maximum(r_ref[...], 0.0)
    t_ref[...] = t
    s1 = jnp.sum(t, axis=0, keepdims=True)
    s2 = jnp.sum(t * t, axis=0, keepdims=True)
    st = jnp.concatenate([s1, s2], axis=0)

    @pl.when(j == 0)
    def _():
        stats_ref[...] = st

    @pl.when(j > 0)
    def _():
        stats_ref[...] += st


_post = pl.pallas_call(
    _post_body,
    grid=(N_NODES // _RB,),
    in_specs=[
        pl.BlockSpec((_RB, HH), lambda i: (i, 0)),
        pl.BlockSpec((_RB, HH), lambda i: (i, 0)),
        pl.BlockSpec((_RB, H), lambda i: (i, 0)),
    ],
    out_specs=[
        pl.BlockSpec((_RB, H), lambda i: (i, 0)),
        pl.BlockSpec((2, H), lambda i: (0, 0)),
    ],
    out_shape=[
        jax.ShapeDtypeStruct((N_NODES, H), _F32),
        jax.ShapeDtypeStruct((2, H), _F32),
    ],
)


def _bn_body(t_ref, stats_ref, g_ref, b_ref, o_ref):
    inv_n = 1.0 / N_NODES
    mean = stats_ref[0:1] * inv_n
    var = stats_ref[1:2] * inv_n - mean * mean
    scale = lax.rsqrt(var + 1e-5) * g_ref[...]
    o_ref[...] = (t_ref[...] - mean) * scale + b_ref[...]


_bn = pl.pallas_call(
    _bn_body,
    grid=(N_NODES // _RB,),
    in_specs=[
        pl.BlockSpec((_RB, H), lambda i: (i, 0)),
        pl.BlockSpec((2, H), lambda i: (0, 0)),
        pl.BlockSpec((1, H), lambda i: (0, 0)),
        pl.BlockSpec((1, H), lambda i: (0, 0)),
    ],
    out_specs=pl.BlockSpec((_RB, H), lambda i: (i, 0)),
    out_shape=jax.ShapeDtypeStruct((N_NODES, H), _F32),
)


_HB = 400
_HNB = N_NODES // _HB


def _head_body(h_ref, gid_ref, w1_ref, b1_ref, w2_ref, b2_ref, o_ref, g_ref):
    j = pl.program_id(0)
    oh = (lax.broadcasted_iota(jnp.int32, (N_GRAPHS, _HB), 0)
          == gid_ref[0]).astype(_F32)
    gp = _dot(oh, h_ref[...], precision=_PREC)

    @pl.when(j == 0)
    def _():
        g_ref[...] = gp

    @pl.when(j > 0)
    def _():
        g_ref[...] += gp

    @pl.when(j == _HNB - 1)
    def _():
        a = jnp.maximum(_dot(g_ref[...], w1_ref[...]) + b1_ref[...], 0.0)
        o_ref[...] = _dot(a, w2_ref[...]) + b2_ref[...]


_head = pl.pallas_call(
    _head_body,
    grid=(_HNB,),
    in_specs=[
        pl.BlockSpec((_HB, H), lambda i: (i, 0)),
        pl.BlockSpec((1, 1, _HB), lambda i: (i, 0, 0)),
        pl.BlockSpec((H, 1024), lambda i: (0, 0)),
        pl.BlockSpec((1, 1024), lambda i: (0, 0)),
        pl.BlockSpec((1024, 1), lambda i: (0, 0)),
        pl.BlockSpec((1, 1), lambda i: (0, 0)),
    ],
    out_specs=pl.BlockSpec((N_GRAPHS, 1), lambda i: (0, 0)),
    out_shape=jax.ShapeDtypeStruct((N_GRAPHS, 1), _F32),
    scratch_shapes=[pltpu.VMEM((N_GRAPHS, H), _F32)],
)


def kernel(x, edge_index, graph_ids, W_embed, b_embed, Wg, bg, Wr, br,
           gamma, beta, W1, b1, W2, b2):
    src = edge_index[0].reshape(_NS, _NCH, _K)
    dst = edge_index[1].reshape(_NS, _NCH, _K)
    edges = jnp.stack([src, dst], axis=2)
    edges = edges.reshape(_NS, _NSLAB, _SCH, 2, _K)
    zeros = jnp.zeros((_RPT, HH), _F32)
    gid3 = graph_ids.reshape(_HNB, 1, _HB)

    pad = ((0, 0), (0, 0), (0, HH - HHV))
    Wg0 = jnp.pad(Wg[:, :, :HHV], pad)
    Wg1 = jnp.pad(Wg[:, :, HHV:], pad)
    bpad = ((0, 0), (0, HH - HHV))
    bg0 = jnp.pad(bg[:, :HHV], bpad)
    bg1 = jnp.pad(bg[:, HHV:], bpad)

    h = _embed(x, W_embed, b_embed.reshape(1, H))
    for i in range(N_LAYERS):
        m0, m1, r = _mm2(h, Wg0[i], bg0[i].reshape(1, HH),
                         Wg1[i], bg1[i].reshape(1, HH),
                         Wr[i], br[i].reshape(1, H))
        p0, p1 = _sc_segsum(m0, m1, edges, zeros)
        t, stats = _post(p0, p1, r)
        h = _bn(t, stats, gamma[i].reshape(1, H), beta[i].reshape(1, H))

    return _head(h, gid3, W1, b1.reshape(1, 1024), W2, b2.reshape(1, 1))


# speedup vs baseline: 1.0459x; 1.0459x over previous
import functools

import jax
import jax.numpy as jnp
from jax import lax
from jax.experimental import pallas as pl
from jax.experimental.pallas import tpu as pltpu
from jax.experimental.pallas import tpu_sc as plsc

N_NODES = 10000
N_EDGES = 320000
N_GRAPHS = 64
D_IN = 128
H = 200
HH = 104
HHV = H // 2
N_LAYERS = 5

_F32 = jnp.float32
_PREC = jax.lax.Precision.HIGHEST

_NS = 16
_EPT = N_EDGES // _NS
_K = 100
_NCH = _EPT // _K
_SCH = 20
_NSLAB = _NCH // _SCH
_NBUF = 5
_PAD_NODES = 10112
_RPT = _PAD_NODES // _NS


def _dot(a, b, precision=None):
    return lax.dot_general(a, b, (((1,), (0,)), ((), ())),
                           precision=precision, preferred_element_type=_F32)


def _sc_segsum_body(m0_hbm, m1_hbm, e_hbm, z_hbm, o0_hbm, o1_hbm,
                    islab0, islab1, rows0, rows1, rows2, rows3, rows4, acc,
                    isem0, isem1, gsem0, gsem1, gsem2, gsem3, gsem4,
                    ssem0, ssem1, ssem2, ssem3, ssem4):
    c = lax.axis_index("c")
    s = lax.axis_index("s")
    ib = (islab0, islab1)
    isem = (isem0, isem1)
    rows = (rows0, rows1, rows2, rows3, rows4)
    gs = (gsem0, gsem1, gsem2, gsem3, gsem4)
    ss = (ssem0, ssem1, ssem2, ssem3, ssem4)

    def fire_islab(u):
        pltpu.async_copy(e_hbm.at[s, u], ib[u % 2], isem[u % 2])

    def wait_islab(u):
        pltpu.make_async_copy(e_hbm.at[s, u], ib[u % 2], isem[u % 2]).wait()

    pltpu.sync_copy(z_hbm, acc.at[pl.ds(s * _RPT, _RPT)])
    fire_islab(0)
    fire_islab(1)
    plsc.subcore_barrier()

    def _half(m_hbm, o_hbm):
        def fire_gather(buf, kk, b):
            pltpu.async_copy(m_hbm.at[buf.at[kk, 0]], rows[b], gs[b])

        def wait_gather(buf, kk, b):
            pltpu.make_async_copy(m_hbm.at[buf.at[kk, 0]], rows[b],
                                  gs[b]).wait()

        def fire_scatter(buf, kk, b):
            pltpu.async_copy(rows[b], acc.at[buf.at[kk, 1]], ss[b], add=True)

        def wait_scatter(buf, kk, b):
            pltpu.make_async_copy(rows[b], acc.at[buf.at[kk, 1]],
                                  ss[b]).wait()

        wait_islab(0)
        fire_gather(ib[0], 0, 0)
        fire_gather(ib[0], 1, 1)

        for u in range(_NSLAB):
            bu = ib[u % 2]

            def body(k, carry, u=u, bu=bu):
                kk0 = _NBUF * k
                for b in range(_NBUF):
                    kk = kk0 + b
                    b2 = (b + 2) % _NBUF
                    wait_gather(bu, kk, b)
                    fire_scatter(bu, kk, b)
                    if u == 0 and b < 3:
                        @pl.when(k > 0)
                        def _():
                            wait_scatter(bu, kk, b2)
                    else:
                        wait_scatter(bu, kk, b2)
                    if b < _NBUF - 2:
                        fire_gather(bu, kk + 2, b2)
                    else:
                        @pl.when(k < _SCH // _NBUF - 1)
                        def _():
                            fire_gather(bu, kk + 2, b2)
                    if b == _NBUF - 1 and 1 <= u <= _NSLAB - 2:
                        @pl.when(k == 1)
                        def _():
                            fire_islab(u + 1)
                return carry

            lax.fori_loop(0, _SCH // _NBUF, body, 0)

            if u < _NSLAB - 1:
                nb = ib[(u + 1) % 2]
                wait_islab(u + 1)
                fire_gather(nb, 0, 0)
                fire_gather(nb, 1, 1)

        lastb = ib[(_NSLAB - 1) % 2]
        wait_scatter(lastb, _SCH - 3, (_SCH - 3) % _NBUF)
        wait_scatter(lastb, _SCH - 2, (_SCH - 2) % _NBUF)
        wait_scatter(lastb, _SCH - 1, (_SCH - 1) % _NBUF)
        plsc.subcore_barrier()
        pltpu.sync_copy(acc.at[pl.ds(s * _RPT, _RPT)],
                        o_hbm.at[pl.ds(s * _RPT, _RPT)])

    @pl.when(c == 0)
    def _():
        _half(m0_hbm, o0_hbm)

    @pl.when(c == 1)
    def _():
        _half(m1_hbm, o1_hbm)


_sc_segsum = functools.partial(
    pl.kernel,
    mesh=plsc.VectorSubcoreMesh(core_axis_name="c", subcore_axis_name="s"),
    compiler_params=pltpu.CompilerParams(use_tc_tiling_on_sc=False),
    out_type=[
        jax.ShapeDtypeStruct((_PAD_NODES, HH), _F32),
        jax.ShapeDtypeStruct((_PAD_NODES, HH), _F32),
    ],
    scratch_types=[
        pltpu.VMEM((_SCH, 2, _K), jnp.int32),
        pltpu.VMEM((_SCH, 2, _K), jnp.int32),
        pltpu.VMEM((_K, HH), _F32),
        pltpu.VMEM((_K, HH), _F32),
        pltpu.VMEM((_K, HH), _F32),
        pltpu.VMEM((_K, HH), _F32),
        pltpu.VMEM((_K, HH), _F32),
        pltpu.VMEM_SHARED((_PAD_NODES, HH), _F32),
    ] + [pltpu.SemaphoreType.DMA] * 12,
)(_sc_segsum_body)


_RB = 1000


def _embed_body(x_ref, w_ref, b_ref, o_ref):
    o_ref[...] = _dot(x_ref[...], w_ref[...]) + b_ref[...]


_embed = pl.pallas_call(
    _embed_body,
    grid=(N_NODES // _RB,),
    in_specs=[
        pl.BlockSpec((_RB, D_IN), lambda i: (i, 0)),
        pl.BlockSpec((D_IN, H), lambda i: (0, 0)),
        pl.BlockSpec((1, H), lambda i: (0, 0)),
    ],
    out_specs=pl.BlockSpec((_RB, H), lambda i: (i, 0)),
    out_shape=jax.ShapeDtypeStruct((N_NODES, H), _F32),
)


def _affine(t, stats_ref, g_ref, b_ref):
    inv_n = 1.0 / N_NODES
    mean = stats_ref[0:1] * inv_n
    var = stats_ref[1:2] * inv_n - mean * mean
    scale = lax.rsqrt(var + 1e-5) * g_ref[...]
    return (t - mean) * scale + b_ref[...]


def _mmg_body(t_ref, stats_ref, g_ref, b_ref,
              wg0_ref, bg0_ref, wg1_ref, bg1_ref, m0_ref, m1_ref):
    h = _affine(t_ref[...], stats_ref, g_ref, b_ref)
    m0_ref[...] = _dot(h, wg0_ref[...]) + bg0_ref[...]
    m1_ref[...] = _dot(h, wg1_ref[...]) + bg1_ref[...]


_mmg = pl.pallas_call(
    _mmg_body,
    grid=(N_NODES // _RB,),
    in_specs=[
        pl.BlockSpec((_RB, H), lambda i: (i, 0)),
        pl.BlockSpec((2, H), lambda i: (0, 0)),
        pl.BlockSpec((1, H), lambda i: (0, 0)),
        pl.BlockSpec((1, H), lambda i: (0, 0)),
        pl.BlockSpec((H, HH), lambda i: (0, 0)),
        pl.BlockSpec((1, HH), lambda i: (0, 0)),
        pl.BlockSpec((H, HH), lambda i: (0, 0)),
        pl.BlockSpec((1, HH), lambda i: (0, 0)),
    ],
    out_specs=[
        pl.BlockSpec((_RB, HH), lambda i: (i, 0)),
        pl.BlockSpec((_RB, HH), lambda i: (i, 0)),
    ],
    out_shape=[
        jax.ShapeDtypeStruct((N_NODES, HH), _F32),
        jax.ShapeDtypeStruct((N_NODES, HH), _F32),
    ],
)


def _mmr_body(t_ref, stats_ref, g_ref, b_ref, wr_ref, br_ref, r_ref):
    h = _affine(t_ref[...], stats_ref, g_ref, b_ref)
    r_ref[...] = _dot(h, wr_ref[...]) + br_ref[...]


_mmr = pl.pallas_call(
    _mmr_body,
    grid=(N_NODES // _RB,),
    in_specs=[
        pl.BlockSpec((_RB, H), lambda i: (i, 0)),
        pl.BlockSpec((2, H), lambda i: (0, 0)),
        pl.BlockSpec((1, H), lambda i: (0, 0)),
        pl.BlockSpec((1, H), lambda i: (0, 0)),
        pl.BlockSpec((H, H), lambda i: (0, 0)),
        pl.BlockSpec((1, H), lambda i: (0, 0)),
    ],
    out_specs=pl.BlockSpec((_RB, H), lambda i: (i, 0)),
    out_shape=jax.ShapeDtypeStruct((N_NODES, H), _F32),
)


def _post_body(p0_ref, p1_ref, r_ref, t_ref, stats_ref):
    j = pl.program_id(0)
    a = jnp.concatenate([p0_ref[:, :HHV], p1_ref[:, :HHV]], axis=1)
    t = jnp.maximum(a, 0.0) + jnp.maximum(r_ref[...], 0.0)
    t_ref[...] = t
    s1 = jnp.sum(t, axis=0, keepdims=True)
    s2 = jnp.sum(t * t, axis=0, keepdims=True)
    st = jnp.concatenate([s1, s2], axis=0)

    @pl.when(j == 0)
    def _():
        stats_ref[...] = st

    @pl.when(j > 0)
    def _():
        stats_ref[...] += st


_post = pl.pallas_call(
    _post_body,
    grid=(N_NODES // _RB,),
    in_specs=[
        pl.BlockSpec((_RB, HH), lambda i: (i, 0)),
        pl.BlockSpec((_RB, HH), lambda i: (i, 0)),
        pl.BlockSpec((_RB, H), lambda i: (i, 0)),
    ],
    out_specs=[
        pl.BlockSpec((_RB, H), lambda i: (i, 0)),
        pl.BlockSpec((2, H), lambda i: (0, 0)),
    ],
    out_shape=[
        jax.ShapeDtypeStruct((N_NODES, H), _F32),
        jax.ShapeDtypeStruct((2, H), _F32),
    ],
)


_HB = 400
_HNB = N_NODES // _HB


def _head_body(t_ref, stats_ref, ga_ref, be_ref, gid_ref, w1_ref, b1_ref,
               w2_ref, b2_ref, o_ref, g_ref):
    j = pl.program_id(0)
    h = _affine(t_ref[...], stats_ref, ga_ref, be_ref)
    oh = (lax.broadcasted_iota(jnp.int32, (N_GRAPHS, _HB), 0)
          == gid_ref[0]).astype(_F32)
    gp = _dot(oh, h, precision=_PREC)

    @pl.when(j == 0)
    def _():
        g_ref[...] = gp

    @pl.when(j > 0)
    def _():
        g_ref[...] += gp

    @pl.when(j == _HNB - 1)
    def _():
        a = jnp.maximum(_dot(g_ref[...], w1_ref[...]) + b1_ref[...], 0.0)
        o_ref[...] = _dot(a, w2_ref[...]) + b2_ref[...]


_head = pl.pallas_call(
    _head_body,
    grid=(_HNB,),
    in_specs=[
        pl.BlockSpec((_HB, H), lambda i: (i, 0)),
        pl.BlockSpec((2, H), lambda i: (0, 0)),
        pl.BlockSpec((1, H), lambda i: (0, 0)),
        pl.BlockSpec((1, H), lambda i: (0, 0)),
        pl.BlockSpec((1, 1, _HB), lambda i: (i, 0, 0)),
        pl.BlockSpec((H, 1024), lambda i: (0, 0)),
        pl.BlockSpec((1, 1024), lambda i: (0, 0)),
        pl.BlockSpec((1024, 1), lambda i: (0, 0)),
        pl.BlockSpec((1, 1), lambda i: (0, 0)),
    ],
    out_specs=pl.BlockSpec((N_GRAPHS, 1), lambda i: (0, 0)),
    out_shape=jax.ShapeDtypeStruct((N_GRAPHS, 1), _F32),
    scratch_shapes=[pltpu.VMEM((N_GRAPHS, H), _F32)],
)


def kernel(x, edge_index, graph_ids, W_embed, b_embed, Wg, bg, Wr, br,
           gamma, beta, W1, b1, W2, b2):
    src = edge_index[0].reshape(_NS, _NCH, _K)
    dst = edge_index[1].reshape(_NS, _NCH, _K)
    edges = jnp.stack([src, dst], axis=2)
    edges = edges.reshape(_NS, _NSLAB, _SCH, 2, _K)
    zeros = jnp.zeros((_RPT, HH), _F32)
    gid3 = graph_ids.reshape(_HNB, 1, _HB)

    pad = ((0, 0), (0, 0), (0, HH - HHV))
    Wg0 = jnp.pad(Wg[:, :, :HHV], pad)
    Wg1 = jnp.pad(Wg[:, :, HHV:], pad)
    bpad = ((0, 0), (0, HH - HHV))
    bg0 = jnp.pad(bg[:, :HHV], bpad)
    bg1 = jnp.pad(bg[:, HHV:], bpad)

    t = _embed(x, W_embed, b_embed.reshape(1, H))
    st = jnp.concatenate([jnp.zeros((1, H), _F32),
                          jnp.full((1, H), N_NODES * (1.0 - 1e-5), _F32)], 0)
    g_aff = jnp.ones((1, H), _F32)
    b_aff = jnp.zeros((1, H), _F32)
    for i in range(N_LAYERS):
        m0, m1 = _mmg(t, st, g_aff, b_aff,
                      Wg0[i], bg0[i].reshape(1, HH),
                      Wg1[i], bg1[i].reshape(1, HH))
        p0, p1 = _sc_segsum(m0, m1, edges, zeros)
        r = _mmr(t, st, g_aff, b_aff, Wr[i], br[i].reshape(1, H))
        t, st = _post(p0, p1, r)
        g_aff = gamma[i].reshape(1, H)
        b_aff = beta[i].reshape(1, H)

    return _head(t, st, g_aff, b_aff, gid3,
                 W1, b1.reshape(1, 1024), W2, b2.reshape(1, 1))


# speedup vs baseline: 1.0514x; 1.0052x over previous
import functools

import jax
import jax.numpy as jnp
from jax import lax
from jax.experimental import pallas as pl
from jax.experimental.pallas import tpu as pltpu
from jax.experimental.pallas import tpu_sc as plsc

N_NODES = 10000
N_EDGES = 320000
N_GRAPHS = 64
D_IN = 128
H = 200
HH = 104
HHV = H // 2
N_LAYERS = 5

_F32 = jnp.float32
_PREC = jax.lax.Precision.HIGHEST

_NS = 16
_EPT = N_EDGES // _NS
_K = 100
_NCH = _EPT // _K
_SCH = 20
_NSLAB = _NCH // _SCH
_NBUF = 5
_PAD_NODES = 10112
_RPT = _PAD_NODES // _NS


def _dot(a, b, precision=None):
    return lax.dot_general(a, b, (((1,), (0,)), ((), ())),
                           precision=precision, preferred_element_type=_F32)


def _sc_segsum_body(m0_hbm, m1_hbm, e_hbm, z_hbm, o0_hbm, o1_hbm,
                    islab0, islab1, rows0, rows1, rows2, rows3, rows4, acc,
                    isem0, isem1, gsem0, gsem1, gsem2, gsem3, gsem4,
                    ssem0, ssem1, ssem2, ssem3, ssem4):
    c = lax.axis_index("c")
    s = lax.axis_index("s")
    ib = (islab0, islab1)
    isem = (isem0, isem1)
    rows = (rows0, rows1, rows2, rows3, rows4)
    gs = (gsem0, gsem1, gsem2, gsem3, gsem4)
    ss = (ssem0, ssem1, ssem2, ssem3, ssem4)

    def fire_islab(u):
        pltpu.async_copy(e_hbm.at[s, u], ib[u % 2], isem[u % 2])

    def wait_islab(u):
        pltpu.make_async_copy(e_hbm.at[s, u], ib[u % 2], isem[u % 2]).wait()

    pltpu.sync_copy(z_hbm, acc.at[pl.ds(s * _RPT, _RPT)])
    fire_islab(0)
    fire_islab(1)
    plsc.subcore_barrier()

    def _half(m_hbm, o_hbm):
        def fire_gather(buf, kk, b):
            pltpu.async_copy(m_hbm.at[buf.at[kk, 0]], rows[b], gs[b])

        def wait_gather(buf, kk, b):
            pltpu.make_async_copy(m_hbm.at[buf.at[kk, 0]], rows[b],
                                  gs[b]).wait()

        def fire_scatter(buf, kk, b):
            pltpu.async_copy(rows[b], acc.at[buf.at[kk, 1]], ss[b], add=True)

        def wait_scatter(buf, kk, b):
            pltpu.make_async_copy(rows[b], acc.at[buf.at[kk, 1]],
                                  ss[b]).wait()

        wait_islab(0)
        fire_gather(ib[0], 0, 0)
        fire_gather(ib[0], 1, 1)

        for u in range(_NSLAB):
            bu = ib[u % 2]

            def body(k, carry, u=u, bu=bu):
                kk0 = _NBUF * k
                for b in range(_NBUF):
                    kk = kk0 + b
                    b2 = (b + 2) % _NBUF
                    wait_gather(bu, kk, b)
                    fire_scatter(bu, kk, b)
                    if u == 0 and b < 3:
                        @pl.when(k > 0)
                        def _():
                            wait_scatter(bu, kk, b2)
                    else:
                        wait_scatter(bu, kk, b2)
                    if b < _NBUF - 2:
                        fire_gather(bu, kk + 2, b2)
                    else:
                        @pl.when(k < _SCH // _NBUF - 1)
                        def _():
                            fire_gather(bu, kk + 2, b2)
                    if b == _NBUF - 1 and 1 <= u <= _NSLAB - 2:
                        @pl.when(k == 1)
                        def _():
                            fire_islab(u + 1)
                return carry

            lax.fori_loop(0, _SCH // _NBUF, body, 0)

            if u < _NSLAB - 1:
                nb = ib[(u + 1) % 2]
                wait_islab(u + 1)
                fire_gather(nb, 0, 0)
                fire_gather(nb, 1, 1)

        lastb = ib[(_NSLAB - 1) % 2]
        wait_scatter(lastb, _SCH - 3, (_SCH - 3) % _NBUF)
        wait_scatter(lastb, _SCH - 2, (_SCH - 2) % _NBUF)
        wait_scatter(lastb, _SCH - 1, (_SCH - 1) % _NBUF)
        plsc.subcore_barrier()
        pltpu.sync_copy(acc.at[pl.ds(s * _RPT, _RPT)],
                        o_hbm.at[pl.ds(s * _RPT, _RPT)])

    @pl.when(c == 0)
    def _():
        _half(m0_hbm, o0_hbm)

    @pl.when(c == 1)
    def _():
        _half(m1_hbm, o1_hbm)


_sc_segsum = functools.partial(
    pl.kernel,
    mesh=plsc.VectorSubcoreMesh(core_axis_name="c", subcore_axis_name="s"),
    compiler_params=pltpu.CompilerParams(use_tc_tiling_on_sc=False),
    out_type=[
        jax.ShapeDtypeStruct((_PAD_NODES, HH), _F32),
        jax.ShapeDtypeStruct((_PAD_NODES, HH), _F32),
    ],
    scratch_types=[
        pltpu.VMEM((_SCH, 2, _K), jnp.int32),
        pltpu.VMEM((_SCH, 2, _K), jnp.int32),
        pltpu.VMEM((_K, HH), _F32),
        pltpu.VMEM((_K, HH), _F32),
        pltpu.VMEM((_K, HH), _F32),
        pltpu.VMEM((_K, HH), _F32),
        pltpu.VMEM((_K, HH), _F32),
        pltpu.VMEM_SHARED((_PAD_NODES, HH), _F32),
    ] + [pltpu.SemaphoreType.DMA] * 12,
)(_sc_segsum_body)


_RB = 1000


def _affine(t, stats_ref, g_ref, b_ref):
    inv_n = 1.0 / N_NODES
    mean = stats_ref[0:1] * inv_n
    var = stats_ref[1:2] * inv_n - mean * mean
    scale = lax.rsqrt(var + 1e-5) * g_ref[...]
    return (t - mean) * scale + b_ref[...]


def _mm_body(t_ref, stats_ref, g_ref, b_ref,
             wg0_ref, bg0_ref, wg1_ref, bg1_ref, wr_ref, br_ref,
             m0_ref, m1_ref, r_ref):
    h = _affine(t_ref[...], stats_ref, g_ref, b_ref)
    m0_ref[...] = _dot(h, wg0_ref[...]) + bg0_ref[...]
    m1_ref[...] = _dot(h, wg1_ref[...]) + bg1_ref[...]
    r_ref[...] = _dot(h, wr_ref[...]) + br_ref[...]


_mm = pl.pallas_call(
    _mm_body,
    grid=(N_NODES // _RB,),
    in_specs=[
        pl.BlockSpec((_RB, H), lambda i: (i, 0)),
        pl.BlockSpec((2, H), lambda i: (0, 0)),
        pl.BlockSpec((1, H), lambda i: (0, 0)),
        pl.BlockSpec((1, H), lambda i: (0, 0)),
        pl.BlockSpec((H, HH), lambda i: (0, 0)),
        pl.BlockSpec((1, HH), lambda i: (0, 0)),
        pl.BlockSpec((H, HH), lambda i: (0, 0)),
        pl.BlockSpec((1, HH), lambda i: (0, 0)),
        pl.BlockSpec((H, H), lambda i: (0, 0)),
        pl.BlockSpec((1, H), lambda i: (0, 0)),
    ],
    out_specs=[
        pl.BlockSpec((_RB, HH), lambda i: (i, 0)),
        pl.BlockSpec((_RB, HH), lambda i: (i, 0)),
        pl.BlockSpec((_RB, H), lambda i: (i, 0)),
    ],
    out_shape=[
        jax.ShapeDtypeStruct((N_NODES, HH), _F32),
        jax.ShapeDtypeStruct((N_NODES, HH), _F32),
        jax.ShapeDtypeStruct((N_NODES, H), _F32),
    ],
)


def _mm0_body(x_ref, we_ref, be_ref,
              wg0_ref, bg0_ref, wg1_ref, bg1_ref, wr_ref, br_ref,
              m0_ref, m1_ref, r_ref):
    h = _dot(x_ref[...], we_ref[...]) + be_ref[...]
    m0_ref[...] = _dot(h, wg0_ref[...]) + bg0_ref[...]
    m1_ref[...] = _dot(h, wg1_ref[...]) + bg1_ref[...]
    r_ref[...] = _dot(h, wr_ref[...]) + br_ref[...]


_mm0 = pl.pallas_call(
    _mm0_body,
    grid=(N_NODES // _RB,),
    in_specs=[
        pl.BlockSpec((_RB, D_IN), lambda i: (i, 0)),
        pl.BlockSpec((D_IN, H), lambda i: (0, 0)),
        pl.BlockSpec((1, H), lambda i: (0, 0)),
        pl.BlockSpec((H, HH), lambda i: (0, 0)),
        pl.BlockSpec((1, HH), lambda i: (0, 0)),
        pl.BlockSpec((H, HH), lambda i: (0, 0)),
        pl.BlockSpec((1, HH), lambda i: (0, 0)),
        pl.BlockSpec((H, H), lambda i: (0, 0)),
        pl.BlockSpec((1, H), lambda i: (0, 0)),
    ],
    out_specs=[
        pl.BlockSpec((_RB, HH), lambda i: (i, 0)),
        pl.BlockSpec((_RB, HH), lambda i: (i, 0)),
        pl.BlockSpec((_RB, H), lambda i: (i, 0)),
    ],
    out_shape=[
        jax.ShapeDtypeStruct((N_NODES, HH), _F32),
        jax.ShapeDtypeStruct((N_NODES, HH), _F32),
        jax.ShapeDtypeStruct((N_NODES, H), _F32),
    ],
)


def _post_body(p0_ref, p1_ref, r_ref, t_ref, stats_ref):
    j = pl.program_id(0)
    a = jnp.concatenate([p0_ref[:, :HHV], p1_ref[:, :HHV]], axis=1)
    t = jnp.maximum(a, 0.0) + jnp.maximum(r_ref[...], 0.0)
    t_ref[...] = t
    s1 = jnp.sum(t, axis=0, keepdims=True)
    s2 = jnp.sum(t * t, axis=0, keepdims=True)
    st = jnp.concatenate([s1, s2], axis=0)

    @pl.when(j == 0)
    def _():
        stats_ref[...] = st

    @pl.when(j > 0)
    def _():
        stats_ref[...] += st


_post = pl.pallas_call(
    _post_body,
    grid=(N_NODES // _RB,),
    in_specs=[
        pl.BlockSpec((_RB, HH), lambda i: (i, 0)),
        pl.BlockSpec((_RB, HH), lambda i: (i, 0)),
        pl.BlockSpec((_RB, H), lambda i: (i, 0)),
    ],
    out_specs=[
        pl.BlockSpec((_RB, H), lambda i: (i, 0)),
        pl.BlockSpec((2, H), lambda i: (0, 0)),
    ],
    out_shape=[
        jax.ShapeDtypeStruct((N_NODES, H), _F32),
        jax.ShapeDtypeStruct((2, H), _F32),
    ],
)


_HB = 400
_HNB = N_NODES // _HB


def _head_body(t_ref, stats_ref, ga_ref, be_ref, gid_ref, w1_ref, b1_ref,
               w2_ref, b2_ref, o_ref, g_ref):
    j = pl.program_id(0)
    h = _affine(t_ref[...], stats_ref, ga_ref, be_ref)
    oh = (lax.broadcasted_iota(jnp.int32, (N_GRAPHS, _HB), 0)
          == gid_ref[0]).astype(_F32)
    gp = _dot(oh, h, precision=_PREC)

    @pl.when(j == 0)
    def _():
        g_ref[...] = gp

    @pl.when(j > 0)
    def _():
        g_ref[...] += gp

    @pl.when(j == _HNB - 1)
    def _():
        a = jnp.maximum(_dot(g_ref[...], w1_ref[...]) + b1_ref[...], 0.0)
        o_ref[...] = _dot(a, w2_ref[...]) + b2_ref[...]


_head = pl.pallas_call(
    _head_body,
    grid=(_HNB,),
    in_specs=[
        pl.BlockSpec((_HB, H), lambda i: (i, 0)),
        pl.BlockSpec((2, H), lambda i: (0, 0)),
        pl.BlockSpec((1, H), lambda i: (0, 0)),
        pl.BlockSpec((1, H), lambda i: (0, 0)),
        pl.BlockSpec((1, 1, _HB), lambda i: (i, 0, 0)),
        pl.BlockSpec((H, 1024), lambda i: (0, 0)),
        pl.BlockSpec((1, 1024), lambda i: (0, 0)),
        pl.BlockSpec((1024, 1), lambda i: (0, 0)),
        pl.BlockSpec((1, 1), lambda i: (0, 0)),
    ],
    out_specs=pl.BlockSpec((N_GRAPHS, 1), lambda i: (0, 0)),
    out_shape=jax.ShapeDtypeStruct((N_GRAPHS, 1), _F32),
    scratch_shapes=[pltpu.VMEM((N_GRAPHS, H), _F32)],
)


def kernel(x, edge_index, graph_ids, W_embed, b_embed, Wg, bg, Wr, br,
           gamma, beta, W1, b1, W2, b2):
    src = edge_index[0].reshape(_NS, _NCH, _K)
    dst = edge_index[1].reshape(_NS, _NCH, _K)
    edges = jnp.stack([src, dst], axis=2)
    edges = edges.reshape(_NS, _NSLAB, _SCH, 2, _K)
    zeros = jnp.zeros((_RPT, HH), _F32)
    gid3 = graph_ids.reshape(_HNB, 1, _HB)

    pad = ((0, 0), (0, 0), (0, HH - HHV))
    Wg0 = jnp.pad(Wg[:, :, :HHV], pad)
    Wg1 = jnp.pad(Wg[:, :, HHV:], pad)
    bpad = ((0, 0), (0, HH - HHV))
    bg0 = jnp.pad(bg[:, :HHV], bpad)
    bg1 = jnp.pad(bg[:, HHV:], bpad)

    t = st = g_aff = b_aff = None
    for i in range(N_LAYERS):
        if i == 0:
            m0, m1, r = _mm0(x, W_embed, b_embed.reshape(1, H),
                             Wg0[i], bg0[i].reshape(1, HH),
                             Wg1[i], bg1[i].reshape(1, HH),
                             Wr[i], br[i].reshape(1, H))
        else:
            m0, m1, r = _mm(t, st, g_aff, b_aff,
                            Wg0[i], bg0[i].reshape(1, HH),
                            Wg1[i], bg1[i].reshape(1, HH),
                            Wr[i], br[i].reshape(1, H))
        p0, p1 = _sc_segsum(m0, m1, edges, zeros)
        t, st = _post(p0, p1, r)
        g_aff = gamma[i].reshape(1, H)
        b_aff = beta[i].reshape(1, H)

    return _head(t, st, g_aff, b_aff, gid3,
                 W1, b1.reshape(1, 1024), W2, b2.reshape(1, 1))


# speedup vs baseline: 1.0647x; 1.0127x over previous
import functools

import jax
import jax.numpy as jnp
from jax import lax
from jax.experimental import pallas as pl
from jax.experimental.pallas import tpu as pltpu
from jax.experimental.pallas import tpu_sc as plsc

N_NODES = 10000
N_EDGES = 320000
N_GRAPHS = 64
D_IN = 128
H = 200
HH = 104
HHV = H // 2
N_LAYERS = 5

_F32 = jnp.float32
_PREC = jax.lax.Precision.HIGHEST

_NS = 16
_EPT = N_EDGES // _NS
_K = 100
_NCH = _EPT // _K
_SCH = 20
_NSLAB = _NCH // _SCH
_NBUF = 5
_PAD_NODES = 10112
_RPT = _PAD_NODES // _NS


def _dot(a, b, precision=None):
    return lax.dot_general(a, b, (((1,), (0,)), ((), ())),
                           precision=precision, preferred_element_type=_F32)


def _sc_segsum_body(m0_hbm, m1_hbm, e_hbm, z_hbm, o0_hbm, o1_hbm,
                    islab0, islab1, rows0, rows1, rows2, rows3, rows4, acc,
                    isem0, isem1, gsem0, gsem1, gsem2, gsem3, gsem4,
                    ssem0, ssem1, ssem2, ssem3, ssem4):
    c = lax.axis_index("c")
    s = lax.axis_index("s")
    ib = (islab0, islab1)
    isem = (isem0, isem1)
    rows = (rows0, rows1, rows2, rows3, rows4)
    gs = (gsem0, gsem1, gsem2, gsem3, gsem4)
    ss = (ssem0, ssem1, ssem2, ssem3, ssem4)

    def fire_islab(u):
        pltpu.async_copy(e_hbm.at[s, u], ib[u % 2], isem[u % 2])

    def wait_islab(u):
        pltpu.make_async_copy(e_hbm.at[s, u], ib[u % 2], isem[u % 2]).wait()

    pltpu.sync_copy(z_hbm, acc.at[pl.ds(s * _RPT, _RPT)])
    fire_islab(0)
    fire_islab(1)
    plsc.subcore_barrier()

    def _half(m_hbm, o_hbm):
        def fire_gather(buf, kk, b):
            pltpu.async_copy(m_hbm.at[buf.at[kk, 0]], rows[b], gs[b])

        def wait_gather(buf, kk, b):
            pltpu.make_async_copy(m_hbm.at[buf.at[kk, 0]], rows[b],
                                  gs[b]).wait()

        def fire_scatter(buf, kk, b):
            pltpu.async_copy(rows[b], acc.at[buf.at[kk, 1]], ss[b], add=True)

        def wait_scatter(buf, kk, b):
            pltpu.make_async_copy(rows[b], acc.at[buf.at[kk, 1]],
                                  ss[b]).wait()

        wait_islab(0)
        fire_gather(ib[0], 0, 0)
        fire_gather(ib[0], 1, 1)

        def slab(u, bu, nxt, nsem):
            def body(k, carry):
                kk0 = _NBUF * k
                for b in range(_NBUF):
                    kk = kk0 + b
                    b2 = (b + 2) % _NBUF
                    wait_gather(bu, kk, b)
                    fire_scatter(bu, kk, b)
                    if b < 3:
                        @pl.when(jnp.logical_or(u > 0, k > 0))
                        def _():
                            wait_scatter(bu, kk, b2)
                    else:
                        wait_scatter(bu, kk, b2)
                    if b < _NBUF - 2:
                        fire_gather(bu, kk + 2, b2)
                    else:
                        @pl.when(k < _SCH // _NBUF - 1)
                        def _():
                            fire_gather(bu, kk + 2, b2)
                    if b == _NBUF - 1:
                        @pl.when(jnp.logical_and(
                            k == 1, jnp.logical_and(u >= 1, u <= _NSLAB - 2)))
                        def _():
                            pltpu.async_copy(e_hbm.at[s, u + 1], nxt, nsem)
                return carry

            lax.fori_loop(0, _SCH // _NBUF, body, 0)

            @pl.when(u < _NSLAB - 1)
            def _():
                pltpu.make_async_copy(e_hbm.at[s, u + 1], nxt, nsem).wait()
                fire_gather(nxt, 0, 0)
                fire_gather(nxt, 1, 1)

        def pair(k2, carry):
            u0 = 2 * k2
            slab(u0, ib[0], ib[1], isem[1])
            slab(u0 + 1, ib[1], ib[0], isem[0])
            return carry

        lax.fori_loop(0, _NSLAB // 2, pair, 0)

        lastb = ib[(_NSLAB - 1) % 2]
        wait_scatter(lastb, _SCH - 3, (_SCH - 3) % _NBUF)
        wait_scatter(lastb, _SCH - 2, (_SCH - 2) % _NBUF)
        wait_scatter(lastb, _SCH - 1, (_SCH - 1) % _NBUF)
        plsc.subcore_barrier()
        pltpu.sync_copy(acc.at[pl.ds(s * _RPT, _RPT)],
                        o_hbm.at[pl.ds(s * _RPT, _RPT)])

    @pl.when(c == 0)
    def _():
        _half(m0_hbm, o0_hbm)

    @pl.when(c == 1)
    def _():
        _half(m1_hbm, o1_hbm)


_sc_segsum = functools.partial(
    pl.kernel,
    mesh=plsc.VectorSubcoreMesh(core_axis_name="c", subcore_axis_name="s"),
    compiler_params=pltpu.CompilerParams(use_tc_tiling_on_sc=False),
    out_type=[
        jax.ShapeDtypeStruct((_PAD_NODES, HH), _F32),
        jax.ShapeDtypeStruct((_PAD_NODES, HH), _F32),
    ],
    scratch_types=[
        pltpu.VMEM((_SCH, 2, _K), jnp.int32),
        pltpu.VMEM((_SCH, 2, _K), jnp.int32),
        pltpu.VMEM((_K, HH), _F32),
        pltpu.VMEM((_K, HH), _F32),
        pltpu.VMEM((_K, HH), _F32),
        pltpu.VMEM((_K, HH), _F32),
        pltpu.VMEM((_K, HH), _F32),
        pltpu.VMEM_SHARED((_PAD_NODES, HH), _F32),
    ] + [pltpu.SemaphoreType.DMA] * 12,
)(_sc_segsum_body)


_RB = 1000


def _affine(t, stats_ref, g_ref, b_ref):
    inv_n = 1.0 / N_NODES
    mean = stats_ref[0:1] * inv_n
    var = stats_ref[1:2] * inv_n - mean * mean
    scale = lax.rsqrt(var + 1e-5) * g_ref[...]
    return (t - mean) * scale + b_ref[...]


def _mm_body(t_ref, stats_ref, g_ref, b_ref,
             wg0_ref, bg0_ref, wg1_ref, bg1_ref, wr_ref, br_ref,
             m0_ref, m1_ref, r_ref):
    h = _affine(t_ref[...], stats_ref, g_ref, b_ref)
    m0_ref[...] = _dot(h, wg0_ref[...]) + bg0_ref[...]
    m1_ref[...] = _dot(h, wg1_ref[...]) + bg1_ref[...]
    r_ref[...] = _dot(h, wr_ref[...]) + br_ref[...]


_mm = pl.pallas_call(
    _mm_body,
    grid=(N_NODES // _RB,),
    in_specs=[
        pl.BlockSpec((_RB, H), lambda i: (i, 0)),
        pl.BlockSpec((2, H), lambda i: (0, 0)),
        pl.BlockSpec((1, H), lambda i: (0, 0)),
        pl.BlockSpec((1, H), lambda i: (0, 0)),
        pl.BlockSpec((H, HH), lambda i: (0, 0)),
        pl.BlockSpec((1, HH), lambda i: (0, 0)),
        pl.BlockSpec((H, HH), lambda i: (0, 0)),
        pl.BlockSpec((1, HH), lambda i: (0, 0)),
        pl.BlockSpec((H, H), lambda i: (0, 0)),
        pl.BlockSpec((1, H), lambda i: (0, 0)),
    ],
    out_specs=[
        pl.BlockSpec((_RB, HH), lambda i: (i, 0)),
        pl.BlockSpec((_RB, HH), lambda i: (i, 0)),
        pl.BlockSpec((_RB, H), lambda i: (i, 0)),
    ],
    out_shape=[
        jax.ShapeDtypeStruct((N_NODES, HH), _F32),
        jax.ShapeDtypeStruct((N_NODES, HH), _F32),
        jax.ShapeDtypeStruct((N_NODES, H), _F32),
    ],
)


def _mm0_body(x_ref, we_ref, be_ref,
              wg0_ref, bg0_ref, wg1_ref, bg1_ref, wr_ref, br_ref,
              m0_ref, m1_ref, r_ref):
    h = _dot(x_ref[...], we_ref[...]) + be_ref[...]
    m0_ref[...] = _dot(h, wg0_ref[...]) + bg0_ref[...]
    m1_ref[...] = _dot(h, wg1_ref[...]) + bg1_ref[...]
    r_ref[...] = _dot(h, wr_ref[...]) + br_ref[...]


_mm0 = pl.pallas_call(
    _mm0_body,
    grid=(N_NODES // _RB,),
    in_specs=[
        pl.BlockSpec((_RB, D_IN), lambda i: (i, 0)),
        pl.BlockSpec((D_IN, H), lambda i: (0, 0)),
        pl.BlockSpec((1, H), lambda i: (0, 0)),
        pl.BlockSpec((H, HH), lambda i: (0, 0)),
        pl.BlockSpec((1, HH), lambda i: (0, 0)),
        pl.BlockSpec((H, HH), lambda i: (0, 0)),
        pl.BlockSpec((1, HH), lambda i: (0, 0)),
        pl.BlockSpec((H, H), lambda i: (0, 0)),
        pl.BlockSpec((1, H), lambda i: (0, 0)),
    ],
    out_specs=[
        pl.BlockSpec((_RB, HH), lambda i: (i, 0)),
        pl.BlockSpec((_RB, HH), lambda i: (i, 0)),
        pl.BlockSpec((_RB, H), lambda i: (i, 0)),
    ],
    out_shape=[
        jax.ShapeDtypeStruct((N_NODES, HH), _F32),
        jax.ShapeDtypeStruct((N_NODES, HH), _F32),
        jax.ShapeDtypeStruct((N_NODES, H), _F32),
    ],
)


def _post_body(p0_ref, p1_ref, r_ref, t_ref, stats_ref):
    j = pl.program_id(0)
    a = jnp.concatenate([p0_ref[:, :HHV], p1_ref[:, :HHV]], axis=1)
    t = jnp.maximum(a, 0.0) + jnp.maximum(r_ref[...], 0.0)
    t_ref[...] = t
    s1 = jnp.sum(t, axis=0, keepdims=True)
    s2 = jnp.sum(t * t, axis=0, keepdims=True)
    st = jnp.concatenate([s1, s2], axis=0)

    @pl.when(j == 0)
    def _():
        stats_ref[...] = st

    @pl.when(j > 0)
    def _():
        stats_ref[...] += st


_post = pl.pallas_call(
    _post_body,
    grid=(N_NODES // _RB,),
    in_specs=[
        pl.BlockSpec((_RB, HH), lambda i: (i, 0)),
        pl.BlockSpec((_RB, HH), lambda i: (i, 0)),
        pl.BlockSpec((_RB, H), lambda i: (i, 0)),
    ],
    out_specs=[
        pl.BlockSpec((_RB, H), lambda i: (i, 0)),
        pl.BlockSpec((2, H), lambda i: (0, 0)),
    ],
    out_shape=[
        jax.ShapeDtypeStruct((N_NODES, H), _F32),
        jax.ShapeDtypeStruct((2, H), _F32),
    ],
)


_HB = 400
_HNB = N_NODES // _HB


def _head_body(t_ref, stats_ref, ga_ref, be_ref, gid_ref, w1_ref, b1_ref,
               w2_ref, b2_ref, o_ref, g_ref):
    j = pl.program_id(0)
    h = _affine(t_ref[...], stats_ref, ga_ref, be_ref)
    oh = (lax.broadcasted_iota(jnp.int32, (N_GRAPHS, _HB), 0)
          == gid_ref[0]).astype(_F32)
    gp = _dot(oh, h, precision=_PREC)

    @pl.when(j == 0)
    def _():
        g_ref[...] = gp

    @pl.when(j > 0)
    def _():
        g_ref[...] += gp

    @pl.when(j == _HNB - 1)
    def _():
        a = jnp.maximum(_dot(g_ref[...], w1_ref[...]) + b1_ref[...], 0.0)
        o_ref[...] = _dot(a, w2_ref[...]) + b2_ref[...]


_head = pl.pallas_call(
    _head_body,
    grid=(_HNB,),
    in_specs=[
        pl.BlockSpec((_HB, H), lambda i: (i, 0)),
        pl.BlockSpec((2, H), lambda i: (0, 0)),
        pl.BlockSpec((1, H), lambda i: (0, 0)),
        pl.BlockSpec((1, H), lambda i: (0, 0)),
        pl.BlockSpec((1, 1, _HB), lambda i: (i, 0, 0)),
        pl.BlockSpec((H, 1024), lambda i: (0, 0)),
        pl.BlockSpec((1, 1024), lambda i: (0, 0)),
        pl.BlockSpec((1024, 1), lambda i: (0, 0)),
        pl.BlockSpec((1, 1), lambda i: (0, 0)),
    ],
    out_specs=pl.BlockSpec((N_GRAPHS, 1), lambda i: (0, 0)),
    out_shape=jax.ShapeDtypeStruct((N_GRAPHS, 1), _F32),
    scratch_shapes=[pltpu.VMEM((N_GRAPHS, H), _F32)],
)


def kernel(x, edge_index, graph_ids, W_embed, b_embed, Wg, bg, Wr, br,
           gamma, beta, W1, b1, W2, b2):
    src = edge_index[0].reshape(_NS, _NCH, _K)
    dst = edge_index[1].reshape(_NS, _NCH, _K)
    edges = jnp.stack([src, dst], axis=2)
    edges = edges.reshape(_NS, _NSLAB, _SCH, 2, _K)
    zeros = jnp.zeros((_RPT, HH), _F32)
    gid3 = graph_ids.reshape(_HNB, 1, _HB)

    pad = ((0, 0), (0, 0), (0, HH - HHV))
    Wg0 = jnp.pad(Wg[:, :, :HHV], pad)
    Wg1 = jnp.pad(Wg[:, :, HHV:], pad)
    bpad = ((0, 0), (0, HH - HHV))
    bg0 = jnp.pad(bg[:, :HHV], bpad)
    bg1 = jnp.pad(bg[:, HHV:], bpad)

    t = st = g_aff = b_aff = None
    for i in range(N_LAYERS):
        if i == 0:
            m0, m1, r = _mm0(x, W_embed, b_embed.reshape(1, H),
                             Wg0[i], bg0[i].reshape(1, HH),
                             Wg1[i], bg1[i].reshape(1, HH),
                             Wr[i], br[i].reshape(1, H))
        else:
            m0, m1, r = _mm(t, st, g_aff, b_aff,
                            Wg0[i], bg0[i].reshape(1, HH),
                            Wg1[i], bg1[i].reshape(1, HH),
                            Wr[i], br[i].reshape(1, H))
        p0, p1 = _sc_segsum(m0, m1, edges, zeros)
        t, st = _post(p0, p1, r)
        g_aff = gamma[i].reshape(1, H)
        b_aff = beta[i].reshape(1, H)

    return _head(t, st, g_aff, b_aff, gid3,
                 W1, b1.reshape(1, 1024), W2, b2.reshape(1, 1))
